# baseline (speedup 1.0000x reference)
import jax
import jax.numpy as jnp
from jax.experimental import pallas as pl
from jax.experimental.pallas import tpu as pltpu

C_IN, C1, C2 = 3, 16, 32
PLANE_H, PLANE_W = 58, 8
PLANE_ROWS = PLANE_H * PLANE_W
XS_ROWS = 2 * PLANE_ROWS + 16
LANES1 = 16 * C_IN
K1 = 4 * LANES1
GRP = 448
HS_ROWS = GRP + 16
K2 = 16 * C1
H2, W2 = 55, 6


G = 16


def _fwd_kernel(xs_ref, w1_ref, b1_ref, w2_ref, b2_ref, o_ref, hs_ref):
    hs_ref[GRP:HS_ROWS, :] = jnp.zeros((HS_ROWS - GRP, 4 * C1), jnp.bfloat16)
    for g_b in range(G):
        groups = []
        for dh in range(2):
            for dw in range(2):
                views = []
                for ti in range(2):
                    for tj in range(2):
                        p, q = (dh + ti) % 2, (dw + tj) % 2
                        off = ((dh + ti) // 2) * PLANE_W + (dw + tj) // 2
                        start = p * PLANE_ROWS + off
                        views.append(
                            xs_ref[g_b, pl.ds(start, GRP),
                                   q * LANES1:(q + 1) * LANES1])
                groups.append(jnp.concatenate(views, axis=1))
        p1 = jnp.concatenate(groups, axis=0)
        h1 = jnp.dot(p1, w1_ref[...], preferred_element_type=jnp.float32)
        h1 = jnp.maximum(h1 + b1_ref[...], 0.0).astype(jnp.bfloat16)

        for g in range(4):
            hs_ref[0:GRP, g * C1:(g + 1) * C1] = h1[g * GRP:(g + 1) * GRP, :]

        p2 = jnp.concatenate(
            [hs_ref[pl.ds(0, GRP), :],
             hs_ref[pl.ds(1, GRP), :],
             hs_ref[pl.ds(PLANE_W, GRP), :],
             hs_ref[pl.ds(PLANE_W + 1, GRP), :]],
            axis=1)
        out = jnp.dot(p2, w2_ref[...], preferred_element_type=jnp.float32)
        o_ref[g_b] = jnp.maximum(out + b2_ref[...], 0.0).T


def kernel(x, w1, b1, w2, b2):
    B = x.shape[0]

    xp = jnp.pad(x, ((0, 0), (0, 0), (0, 12), (0, 4))).astype(jnp.bfloat16)
    xs = xp.reshape(B, C_IN, PLANE_H, 2, 4, PLANE_W, 2, 4)
    xs = xs.transpose(0, 3, 2, 5, 6, 1, 4, 7)
    xs = xs.reshape(B, 2 * PLANE_ROWS, 2 * LANES1)
    xs = jnp.pad(xs, ((0, 0), (0, XS_ROWS - 2 * PLANE_ROWS), (0, 0)))

    w1g = (w1.reshape(C1, C_IN, 2, 4, 2, 4)
             .transpose(2, 4, 1, 3, 5, 0)
             .reshape(K1, C1).astype(jnp.bfloat16))
    w2g = (w2.reshape(C2, C1, 2, 2, 2, 2)
             .transpose(2, 4, 3, 5, 1, 0)
             .reshape(K2, C2).astype(jnp.bfloat16))
    b1r = b1.reshape(1, C1).astype(jnp.float32)
    b2r = b2.reshape(1, C2).astype(jnp.float32)

    vmem = pl.BlockSpec(memory_space=pltpu.MemorySpace.VMEM)
    out_pad = pl.pallas_call(
        _fwd_kernel,
        out_shape=jax.ShapeDtypeStruct((B, C2, GRP), jnp.float32),
        grid=(B // G,),
        in_specs=[
            pl.BlockSpec((G, XS_ROWS, 2 * LANES1), lambda b: (b, 0, 0)),
            vmem, vmem, vmem, vmem,
        ],
        out_specs=pl.BlockSpec((G, C2, GRP), lambda b: (b, 0, 0)),
        scratch_shapes=[pltpu.VMEM((HS_ROWS, 4 * C1), jnp.bfloat16)],
        compiler_params=pltpu.CompilerParams(
            dimension_semantics=("arbitrary",),
            vmem_limit_bytes=64 * 1024 * 1024),
    )(xs, w1g, b1r, w2g, b2r)

    return out_pad.reshape(B, C2, GRP // PLANE_W, PLANE_W)[:, :, :H2, :W2]


# speedup vs baseline: 1.0141x; 1.0141x over previous
import jax
import jax.numpy as jnp
from jax.experimental import pallas as pl
from jax.experimental.pallas import tpu as pltpu

C_IN, C1, C2 = 3, 16, 32
PLANE_H, PLANE_W = 58, 8
PLANE_ROWS = PLANE_H * PLANE_W
XS_ROWS = 2 * PLANE_ROWS
LANES1 = 16 * C_IN
K1 = 4 * LANES1
GRP = 448
HS_ROWS = GRP + 16
K2 = 16 * C1
H2, W2 = 55, 6


G = 16


def _fwd_kernel(xs_ref, w1_ref, b1_ref, w2_ref, b2_ref, o_ref, hs_ref):
    hs_ref[GRP:HS_ROWS, :] = jnp.zeros((HS_ROWS - GRP, 4 * C1), jnp.bfloat16)
    for g_b in range(G):
        groups = []
        for dh in range(2):
            for dw in range(2):
                views = []
                for ti in range(2):
                    for tj in range(2):
                        p, q = (dh + ti) % 2, (dw + tj) % 2
                        off = ((dh + ti) // 2) * PLANE_W + (dw + tj) // 2
                        start = p * PLANE_ROWS + off
                        views.append(
                            xs_ref[g_b, pl.ds(start, GRP),
                                   q * LANES1:(q + 1) * LANES1])
                groups.append(jnp.concatenate(views, axis=1))
        p1 = jnp.concatenate(groups, axis=0)
        h1 = jnp.dot(p1, w1_ref[...], preferred_element_type=jnp.float32)
        h1 = jnp.maximum(h1 + b1_ref[...], 0.0).astype(jnp.bfloat16)

        for g in range(4):
            hs_ref[0:GRP, g * C1:(g + 1) * C1] = h1[g * GRP:(g + 1) * GRP, :]

        p2 = jnp.concatenate(
            [hs_ref[pl.ds(0, GRP), :],
             hs_ref[pl.ds(1, GRP), :],
             hs_ref[pl.ds(PLANE_W, GRP), :],
             hs_ref[pl.ds(PLANE_W + 1, GRP), :]],
            axis=1)
        out = jnp.dot(p2, w2_ref[...], preferred_element_type=jnp.float32)
        o_ref[g_b] = jnp.maximum(out + b2_ref[...], 0.0).T


def kernel(x, w1, b1, w2, b2):
    B = x.shape[0]

    xp = jnp.pad(x, ((0, 0), (0, 0), (0, 12), (0, 4))).astype(jnp.bfloat16)
    xs = xp.reshape(B, C_IN, PLANE_H, 2, 4, PLANE_W, 2, 4)
    xs = xs.transpose(0, 3, 2, 5, 6, 1, 4, 7)
    xs = xs.reshape(B, XS_ROWS, 2 * LANES1)

    w1g = (w1.reshape(C1, C_IN, 2, 4, 2, 4)
             .transpose(2, 4, 1, 3, 5, 0)
             .reshape(K1, C1).astype(jnp.bfloat16))
    w2g = (w2.reshape(C2, C1, 2, 2, 2, 2)
             .transpose(2, 4, 3, 5, 1, 0)
             .reshape(K2, C2).astype(jnp.bfloat16))
    b1r = b1.reshape(1, C1).astype(jnp.float32)
    b2r = b2.reshape(1, C2).astype(jnp.float32)

    vmem = pl.BlockSpec(memory_space=pltpu.MemorySpace.VMEM)
    out_pad = pl.pallas_call(
        _fwd_kernel,
        out_shape=jax.ShapeDtypeStruct((B, C2, GRP), jnp.float32),
        grid=(B // G,),
        in_specs=[
            pl.BlockSpec((G, XS_ROWS, 2 * LANES1), lambda b: (b, 0, 0)),
            vmem, vmem, vmem, vmem,
        ],
        out_specs=pl.BlockSpec((G, C2, GRP), lambda b: (b, 0, 0)),
        scratch_shapes=[pltpu.VMEM((HS_ROWS, 4 * C1), jnp.bfloat16)],
        compiler_params=pltpu.CompilerParams(
            dimension_semantics=("arbitrary",),
            vmem_limit_bytes=64 * 1024 * 1024),
    )(xs, w1g, b1r, w2g, b2r)

    return out_pad.reshape(B, C2, GRP // PLANE_W, PLANE_W)[:, :, :H2, :W2]
